# Initial kernel scaffold; baseline (speedup 1.0000x reference)
import jax
import jax.numpy as jnp
from jax.experimental import pallas as pl


def kernel(index, attr):
    raise NotImplementedError("write your pallas kernel here")


# speedup vs baseline: 24.8745x; 24.8745x over previous
import functools

import jax
import jax.numpy as jnp
from jax import lax
from jax.experimental import pallas as pl
from jax.experimental.pallas import tpu as pltpu
from jax.experimental.pallas import tpu_sc as plsc

B, L, K, D = 4, 4096, 16, 64
BQ = 256
NQ = L // BQ
BIG = 0x7FFFFFFF

NC, NS = 2, 16
NW = NC * NS
ROWS_PER_W = (B * L * K) // NW
CH = 128
NCH = ROWS_PER_W // CH


def _select_body(q_ref, all_ref, gidx_ref, dist_ref):
    q = q_ref[0]
    allv = all_ref[0]
    d = jnp.abs(q - allv)
    j = lax.broadcasted_iota(jnp.int32, (BQ, L), 1)
    keys = jnp.bitwise_or(jnp.left_shift(d, 12), j)
    mins = []
    for _ in range(K):
        m = jnp.min(keys, axis=1, keepdims=True)
        mins.append(m)
        keys = jnp.where(keys == m, BIG, keys)
    packed = jnp.concatenate(mins, axis=1)
    base = (pl.program_id(0) // NQ) * L
    gidx_ref[0] = jnp.bitwise_and(packed, 4095) + base
    dist_ref[0] = jnp.right_shift(packed, 12).astype(jnp.float32)


def _select(vals):
    q = vals.reshape(B * NQ, BQ, 1)
    allv = vals.reshape(B, 1, L)
    return pl.pallas_call(
        _select_body,
        grid=(B * NQ,),
        in_specs=[
            pl.BlockSpec((1, BQ, 1), lambda g: (g, 0, 0)),
            pl.BlockSpec((1, 1, L), lambda g: (g // NQ, 0, 0)),
        ],
        out_specs=[
            pl.BlockSpec((1, BQ, K), lambda g: (g, 0, 0)),
            pl.BlockSpec((1, BQ, K), lambda g: (g, 0, 0)),
        ],
        out_shape=[
            jax.ShapeDtypeStruct((B * NQ, BQ, K), jnp.int32),
            jax.ShapeDtypeStruct((B * NQ, BQ, K), jnp.float32),
        ],
        compiler_params=pltpu.CompilerParams(
            dimension_semantics=("arbitrary",)),
    )(q, allv)


def _gather_body(table_hbm, idx_hbm, out_hbm, idx_v, rows_v, sem):
    wid = lax.axis_index("s") * NC + lax.axis_index("c")
    base = wid * ROWS_PER_W
    pltpu.sync_copy(idx_hbm.at[wid], idx_v)

    @pl.loop(0, NCH)
    def _(j):
        pltpu.async_copy(table_hbm.at[idx_v.at[j]], rows_v, sem).wait()
        pltpu.sync_copy(rows_v, out_hbm.at[pl.ds(base + j * CH, CH)])


def _gather(table, gidx):
    sc_gather = functools.partial(
        pl.kernel,
        mesh=plsc.VectorSubcoreMesh(core_axis_name="c", subcore_axis_name="s"),
        out_type=jax.ShapeDtypeStruct((B * L * K, D), jnp.float32),
        scratch_types=[
            pltpu.VMEM((NCH, CH), jnp.int32),
            pltpu.VMEM((CH, D), jnp.float32),
            pltpu.SemaphoreType.DMA,
        ],
        compiler_params=pltpu.CompilerParams(use_tc_tiling_on_sc=False),
    )(_gather_body)
    return sc_gather(table, gidx)


def kernel(index, attr):
    vals = index[..., 0].astype(jnp.int32)
    gidx, dist = _select(vals)
    rows = _gather(attr.reshape(B * L, D), gidx.reshape(NW, NCH, CH))
    index_distance = dist.reshape(B, L, K, 1)
    neighbors_attr = rows.reshape(B, L, K, D)
    return (index_distance, neighbors_attr)
